# Initial kernel scaffold; baseline (speedup 1.0000x reference)
import jax
import jax.numpy as jnp
from jax.experimental import pallas as pl


def kernel(indices, weight):
    raise NotImplementedError("write your pallas kernel here")


# speedup vs baseline: 3.4483x; 3.4483x over previous
import functools

import jax
import jax.numpy as jnp
from jax import lax
from jax.experimental import pallas as pl
from jax.experimental.pallas import tpu as pltpu
from jax.experimental.pallas import tpu_sc as plsc

NUM_ROWS = 4096 * 50
DIM = 64
NC = 2
NS = 16
NW = NC * NS
ROWS_PER_W = NUM_ROWS // NW
CHUNK = 800
N_CHUNKS = ROWS_PER_W // CHUNK


def _gather_kernel(idx_hbm, table_hbm, out_hbm, idx_v, rows_v, sem):
    wid = lax.axis_index("s") * NC + lax.axis_index("c")
    base = wid * ROWS_PER_W

    def chunk_body(i, carry):
        off = base + i * CHUNK
        pltpu.sync_copy(idx_hbm.at[pl.ds(off, CHUNK)], idx_v)
        pltpu.async_copy(table_hbm.at[idx_v], rows_v, sem).wait()
        pltpu.sync_copy(rows_v, out_hbm.at[pl.ds(off, CHUNK)])
        return carry

    lax.fori_loop(0, N_CHUNKS, chunk_body, 0)


@jax.jit
def _embedding_lookup(idx_flat, weight):
    k = pl.kernel(
        _gather_kernel,
        out_type=jax.ShapeDtypeStruct((NUM_ROWS, DIM), jnp.float32),
        mesh=plsc.VectorSubcoreMesh(core_axis_name="c", subcore_axis_name="s"),
        scratch_types=[
            pltpu.VMEM((CHUNK,), jnp.int32),
            pltpu.VMEM((CHUNK, DIM), jnp.float32),
            pltpu.SemaphoreType.DMA,
        ],
        compiler_params=pltpu.CompilerParams(use_tc_tiling_on_sc=False),
    )
    return k(idx_flat, weight)


def kernel(indices, weight):
    idx_flat = indices.reshape(NUM_ROWS).astype(jnp.int32)
    out = _embedding_lookup(idx_flat, weight)
    return out.reshape(indices.shape[0], indices.shape[1], DIM)


# speedup vs baseline: 3.4916x; 1.0126x over previous
import functools

import jax
import jax.numpy as jnp
from jax import lax
from jax.experimental import pallas as pl
from jax.experimental.pallas import tpu as pltpu
from jax.experimental.pallas import tpu_sc as plsc

NUM_ROWS = 4096 * 50
DIM = 64
NC = 2
NS = 16
NW = NC * NS
ROWS_PER_W = NUM_ROWS // NW
CHUNK = 800
N_CHUNKS = ROWS_PER_W // CHUNK


def _gather_kernel(idx_hbm, table_hbm, out_hbm,
                   idx0, idx1, rows0, rows1, gsem, wsem0, wsem1):
    wid = lax.axis_index("s") * NC + lax.axis_index("c")
    base = wid * ROWS_PER_W

    idx_v = (idx0, idx1)
    rows_v = (rows0, rows1)
    wsem = (wsem0, wsem1)
    pending_wb = [None, None]

    for i in range(N_CHUNKS):
        b = i & 1
        off = base + i * CHUNK
        if pending_wb[b] is not None:
            pending_wb[b].wait()
        pltpu.sync_copy(idx_hbm.at[pl.ds(off, CHUNK)], idx_v[b])
        pltpu.async_copy(table_hbm.at[idx_v[b]], rows_v[b], gsem).wait()
        pending_wb[b] = pltpu.async_copy(
            rows_v[b], out_hbm.at[pl.ds(off, CHUNK)], wsem[b])

    pending_wb[0].wait()
    pending_wb[1].wait()


@jax.jit
def _embedding_lookup(idx_flat, weight):
    k = pl.kernel(
        _gather_kernel,
        out_type=jax.ShapeDtypeStruct((NUM_ROWS, DIM), jnp.float32),
        mesh=plsc.VectorSubcoreMesh(core_axis_name="c", subcore_axis_name="s"),
        scratch_types=[
            pltpu.VMEM((CHUNK,), jnp.int32),
            pltpu.VMEM((CHUNK,), jnp.int32),
            pltpu.VMEM((CHUNK, DIM), jnp.float32),
            pltpu.VMEM((CHUNK, DIM), jnp.float32),
            pltpu.SemaphoreType.DMA,
            pltpu.SemaphoreType.DMA,
            pltpu.SemaphoreType.DMA,
        ],
        compiler_params=pltpu.CompilerParams(use_tc_tiling_on_sc=False),
    )
    return k(idx_flat, weight)


def kernel(indices, weight):
    idx_flat = indices.reshape(NUM_ROWS).astype(jnp.int32)
    out = _embedding_lookup(idx_flat, weight)
    return out.reshape(indices.shape[0], indices.shape[1], DIM)
